# Initial kernel scaffold; baseline (speedup 1.0000x reference)
import jax
import jax.numpy as jnp
from jax.experimental import pallas as pl


def kernel(x, pools):
    raise NotImplementedError("write your pallas kernel here")


# speedup vs baseline: 4.2289x; 4.2289x over previous
import jax
import jax.numpy as jnp
from jax import lax
from jax.experimental import pallas as pl
from jax.experimental.pallas import tpu as pltpu
from jax.experimental.pallas import tpu_sc as plsc

NC = 2
NS = 16
NW = NC * NS
D = 128
K = 16
ROWS_PER_CHUNK = 8
IDX_PER_CHUNK = ROWS_PER_CHUNK * K
VPR = D // 16


def _body(x_hbm, idx_hbm, out_hbm, idx_v, gat_v, out_v, sem0, sem1):
    wid = lax.axis_index("s") * NC + lax.axis_index("c")
    n_chunks = idx_hbm.shape[1]
    base_row = wid * (n_chunks * ROWS_PER_CHUNK)

    pltpu.sync_copy(idx_hbm.at[wid], idx_v)

    sems = (sem0, sem1)

    def start_gather(c, b, sem):
        return pltpu.async_copy(x_hbm.at[idx_v.at[c]], gat_v.at[b], sem)

    start_gather(0, 0, sem0)
    start_gather(1, 1, sem1)

    def compute_chunk(b):
        def row_step(r, _):
            base = r * K
            for v in range(VPR):
                col = pl.ds(v * 16, 16)
                acc = gat_v[b, base, col]
                for j in range(1, K):
                    acc = jnp.maximum(acc, gat_v[b, base + j, col])
                out_v[r, col] = acc
            return 0

        lax.fori_loop(0, ROWS_PER_CHUNK, row_step, 0, unroll=False)

    def step(g, _):
        for b in range(2):
            c = g * 2 + b
            sem = sems[b]
            pltpu.make_async_copy(x_hbm.at[idx_v.at[c]], gat_v.at[b], sem).wait()
            compute_chunk(b)
            pltpu.sync_copy(
                out_v, out_hbm.at[pl.ds(base_row + c * ROWS_PER_CHUNK,
                                        ROWS_PER_CHUNK)])
            next_c = c + 2

            @pl.when(next_c < n_chunks)
            def _():
                start_gather(next_c, b, sem)

        return 0

    lax.fori_loop(0, n_chunks // 2, step, 0, unroll=False)


def kernel(x, pools):
    n2 = pools.shape[0]
    idx = pools.astype(jnp.int32)

    block = NW * ROWS_PER_CHUNK
    n_pad = ((n2 + block - 1) // block) * block
    if n_pad != n2:
        idx = jnp.pad(idx, ((0, n_pad - n2), (0, 0)))
    rows_per_worker = n_pad // NW
    n_chunks = rows_per_worker // ROWS_PER_CHUNK
    if n_chunks % 2 != 0:
        extra = NW * ROWS_PER_CHUNK
        idx = jnp.pad(idx, ((0, extra), (0, 0)))
        n_pad += extra
        rows_per_worker = n_pad // NW
        n_chunks = rows_per_worker // ROWS_PER_CHUNK

    idx_r = idx.reshape(NW, n_chunks, IDX_PER_CHUNK)

    mesh = plsc.VectorSubcoreMesh(core_axis_name="c", subcore_axis_name="s")
    run = pl.kernel(
        _body,
        out_type=jax.ShapeDtypeStruct((n_pad, D), jnp.float32),
        mesh=mesh,
        scratch_types=[
            pltpu.VMEM((n_chunks, IDX_PER_CHUNK), jnp.int32),
            pltpu.VMEM((2, IDX_PER_CHUNK, D), jnp.float32),
            pltpu.VMEM((ROWS_PER_CHUNK, D), jnp.float32),
            pltpu.SemaphoreType.DMA,
            pltpu.SemaphoreType.DMA,
        ],
    )
    out = run(x, idx_r)
    return out[:n2]
